# baseline (speedup 1.0000x reference)
import jax
import jax.numpy as jnp
from jax import lax
from jax.experimental import pallas as pl
from jax.experimental.pallas import tpu as pltpu
from jax.experimental.pallas import tpu_sc as plsc

H = 64
B = 16384
V = 1000001
NC, NS = 2, 16
NW = NC * NS
CHW = 512
NFULL = 1953
TAILC = NFULL * CHW
GMAX = 62


def _i32(x):
    return jnp.asarray(x, jnp.int32)


def _bc(s):
    return lax.broadcast_in_dim(s, (16,), ())


def _scan_body(tT, lab_hbm, tail_in, out, labv, mylab, mypos, buf0, buf1,
               rowstage, posstage, candcol, candpos, sem0, sem1, ssem):
    w = _i32(lax.axis_index("s") * NC + lax.axis_index("c"))
    nw = jnp.where(w == 0, 62, 61)
    sw = jnp.where(w == 0, 0, 61 * w + 1)
    lo = sw * CHW
    hi = jnp.where(w < NW - 1, (sw + nw) * CHW, _i32(2**30))

    iota = lax.iota(jnp.int32, 16)

    pltpu.async_copy(tT.at[:, pl.ds(pl.multiple_of(lo, CHW), CHW)], buf0, sem0)
    pltpu.async_copy(tT.at[:, pl.ds(pl.multiple_of(lo + CHW, CHW), CHW)],
                     buf1, sem1)

    pltpu.sync_copy(lab_hbm, labv)

    def route_body(v2, n):
        lab0 = labv[pl.ds(v2 * 32, 16)]
        lab1 = labv[pl.ds(v2 * 32 + 16, 16)]
        m0 = (lab0 >= _bc(lo)) & (lab0 < _bc(hi))
        m1 = (lab1 >= _bc(lo)) & (lab1 < _bc(hi))
        cnt0 = jnp.sum(jnp.where(m0, 1, 0))
        cnt1 = jnp.sum(jnp.where(m1, 1, 0))

        def do(n):
            cum0 = plsc.cumsum(jnp.where(m0, 1, 0))
            cum1 = plsc.cumsum(jnp.where(m1, 1, 0))
            d0 = _bc(n) + cum0 - 1
            d1 = _bc(n + cnt0) + cum1 - 1
            plsc.store_scatter(mylab, [d0], lab0, mask=m0)
            plsc.store_scatter(mypos, [d0], _bc(v2 * 32) + iota, mask=m0)
            plsc.store_scatter(mylab, [d1], lab1, mask=m1)
            plsc.store_scatter(mypos, [d1], _bc(v2 * 32 + 16) + iota, mask=m1)
            return n + cnt0 + cnt1

        return lax.cond(cnt0 + cnt1 > 0, do, lambda n: n, n)

    n = lax.fori_loop(0, B // 32, route_body, _i32(0))
    nv = (n + 15) // 16

    def flush16(bufb, col16, p16, fc):
        s = fc % 4

        def drain(_):
            pltpu.make_async_copy(rowstage.at[s], out.at[posstage.at[s]],
                                  ssem).wait()
            return 0

        lax.cond(fc >= 4, drain, lambda _: 0, 0)
        plsc.store_scatter(posstage, [_bc(s), iota], p16)
        for h in range(H):
            vals = plsc.load_gather(bufb, [_bc(_i32(h)), col16])
            plsc.store_scatter(rowstage, [_bc(s), iota, _bc(_i32(h))], vals)
        pltpu.async_copy(rowstage.at[s], out.at[posstage.at[s]], ssem)
        return fc + 1

    RING = 256

    def extract_chunk(bufb, wait_fn, clo, fc):
        def flush_group(st):
            wr, rd, fc, wtd = st
            sel = (_bc(rd) + iota) % RING
            col16 = plsc.load_gather(candcol, [sel])
            p16 = plsc.load_gather(candpos, [sel])
            fc = flush16(bufb, col16, p16, fc)
            return (wr, rd + 16, fc, wtd)

        def ensure_wait(st):
            wr, rd, fc, wtd = st

            def do(_):
                wait_fn()
                return 0

            lax.cond(wtd == 0, do, lambda _: 0, 0)
            return (wr, rd, fc, _i32(1))

        def half(idxu, labg, lane_ok, m, cnt, st):
            def append(st):
                wr, rd, fc, wtd = st
                posv = plsc.load_gather(mypos, [idxu], mask=lane_ok)
                cum = plsc.cumsum(jnp.where(m, 1, 0))
                q = (_bc(wr) + cum - 1) % RING
                plsc.store_scatter(candcol, [q], labg - _bc(clo), mask=m)
                plsc.store_scatter(candpos, [q], posv, mask=m)
                st = (wr + cnt, rd, fc, wtd)

                def overflow(st):
                    return flush_group(ensure_wait(st))

                return lax.cond(st[0] - st[1] >= RING - 32, overflow,
                                lambda st: st, st)

            return lax.cond(cnt > 0, append, lambda st: st, st)

        def u_body(u, st):
            idx0 = _bc(u * 32) + iota
            idx1 = _bc(u * 32 + 16) + iota
            ok0 = idx0 < _bc(n)
            ok1 = idx1 < _bc(n)
            lab0 = plsc.load_gather(mylab, [idx0], mask=ok0)
            lab1 = plsc.load_gather(mylab, [idx1], mask=ok1)
            m0 = ok0 & (lab0 >= _bc(clo)) & (lab0 < _bc(clo + CHW))
            m1 = ok1 & (lab1 >= _bc(clo)) & (lab1 < _bc(clo + CHW))
            cnt0 = jnp.sum(jnp.where(m0, 1, 0))
            cnt1 = jnp.sum(jnp.where(m1, 1, 0))
            st = half(idx0, lab0, ok0, m0, cnt0, st)
            return half(idx1, lab1, ok1, m1, cnt1, st)

        st = (_i32(0), _i32(0), fc, _i32(0))
        st = lax.fori_loop(0, (n + 31) // 32, u_body, st)
        st = ensure_wait(st)

        def fg_body(_, st):
            return flush_group(st)

        wr, rd, fc, wtd = st
        st = lax.fori_loop(0, (wr - rd) // 16, fg_body, st)

        def tail(st):
            wr, rd, fc, wtd = st
            k = wr - rd
            sel = (_bc(rd) + iota) % RING
            col16 = plsc.load_gather(candcol, [sel])
            p16 = plsc.load_gather(candpos, [sel])
            c0 = plsc.load_gather(candcol, [_bc(rd % RING)])
            p0 = plsc.load_gather(candpos, [_bc(rd % RING)])
            col16 = jnp.where(iota < _bc(k), col16, c0)
            p16 = jnp.where(iota < _bc(k), p16, p0)
            fc = flush16(bufb, col16, p16, fc)
            return (wr, wr, fc, wtd)

        wr, rd, fc, wtd = st
        st = lax.cond(wr - rd > 0, tail, lambda st: st, st)
        return st[2]

    def chunk_step(g, bufb, semb, fc):
        base = (sw + g) * CHW

        def wait_fn():
            pltpu.make_async_copy(
                tT.at[:, pl.ds(pl.multiple_of(base, CHW), CHW)],
                bufb, semb).wait()

        fc = extract_chunk(bufb, wait_fn, base, fc)

        base2 = (sw + g + 2) * CHW

        @pl.when(g + 2 < nw)
        def _():
            pltpu.async_copy(
                tT.at[:, pl.ds(pl.multiple_of(base2, CHW), CHW)], bufb, semb)

        return fc

    def go_body(go, fc):
        g0 = go * 2
        fc = lax.cond(g0 < nw,
                      lambda fc: chunk_step(g0, buf0, sem0, fc),
                      lambda fc: fc, fc)
        return lax.cond(g0 + 1 < nw,
                        lambda fc: chunk_step(g0 + 1, buf1, sem1, fc),
                        lambda fc: fc, fc)

    fc = lax.fori_loop(0, GMAX // 2, go_body, _i32(0))

    def do_tail(fc):
        pltpu.sync_copy(tail_in, buf0.at[:, pl.ds(0, 128)])
        return extract_chunk(buf0, lambda: None, _i32(TAILC), fc)

    fc = lax.cond(w == NW - 1, do_tail, lambda fc: fc, fc)

    def fdrain(k, _):
        pltpu.make_async_copy(rowstage.at[k % 4],
                              out.at[posstage.at[k % 4]], ssem).wait()
        return 0

    lax.fori_loop(jnp.maximum(fc - 4, 0), fc, fdrain, 0)


def kernel(labels, embedding_table):
    tT = embedding_table.T
    lab = labels.astype(jnp.int32)
    tail = jnp.pad(embedding_table[TAILC:].T, ((0, 0), (0, 128 - (V - TAILC))))
    call = pl.kernel(
        _scan_body,
        out_type=jax.ShapeDtypeStruct((B, 128), jnp.float32),
        mesh=plsc.VectorSubcoreMesh(core_axis_name="c", subcore_axis_name="s"),
        scratch_types=[
            pltpu.VMEM((B,), jnp.int32),
            pltpu.VMEM((B,), jnp.int32),
            pltpu.VMEM((B,), jnp.int32),
            pltpu.VMEM((H, CHW), jnp.float32),
            pltpu.VMEM((H, CHW), jnp.float32),
            pltpu.VMEM((4, 16, 128), jnp.float32),
            pltpu.VMEM((4, 16), jnp.int32),
            pltpu.VMEM((256,), jnp.int32),
            pltpu.VMEM((256,), jnp.int32),
            pltpu.SemaphoreType.DMA,
            pltpu.SemaphoreType.DMA,
            pltpu.SemaphoreType.DMA,
        ],
        compiler_params=pltpu.CompilerParams(needs_layout_passes=False),
    )
    return call(tT, lab, tail)[:, :H]


# speedup vs baseline: 1.0269x; 1.0269x over previous
import jax
import jax.numpy as jnp
from jax import lax
from jax.experimental import pallas as pl
from jax.experimental.pallas import tpu as pltpu
from jax.experimental.pallas import tpu_sc as plsc

H = 64
B = 16384
V = 1000001
NC, NS = 2, 16
NW = NC * NS
CHW = 512
NFULL = 1953
TAILC = NFULL * CHW
GMAX = 62


def _i32(x):
    return jnp.asarray(x, jnp.int32)


def _bc(s):
    return lax.broadcast_in_dim(s, (16,), ())


def _scan_body(tT, lab_hbm, tail_in, out, labv, mylab, mypos, buf0, buf1,
               rowstage, posstage, candcol, candpos, sem0, sem1, ssem):
    w = _i32(lax.axis_index("s") * NC + lax.axis_index("c"))
    nw = jnp.where(w == 0, 62, 61)
    sw = jnp.where(w == 0, 0, 61 * w + 1)
    lo = sw * CHW
    hi = jnp.where(w < NW - 1, (sw + nw) * CHW, _i32(2**30))

    iota = lax.iota(jnp.int32, 16)

    pltpu.async_copy(tT.at[:, pl.ds(pl.multiple_of(lo, CHW), CHW)], buf0, sem0)
    pltpu.async_copy(tT.at[:, pl.ds(pl.multiple_of(lo + CHW, CHW), CHW)],
                     buf1, sem1)

    pltpu.sync_copy(lab_hbm, labv)

    def route_body(v2, n):
        lab0 = labv[pl.ds(v2 * 32, 16)]
        lab1 = labv[pl.ds(v2 * 32 + 16, 16)]
        m0 = (lab0 >= _bc(lo)) & (lab0 < _bc(hi))
        m1 = (lab1 >= _bc(lo)) & (lab1 < _bc(hi))
        cnt0 = jnp.sum(jnp.where(m0, 1, 0))
        cnt1 = jnp.sum(jnp.where(m1, 1, 0))

        def do(n):
            cum0 = plsc.cumsum(jnp.where(m0, 1, 0))
            cum1 = plsc.cumsum(jnp.where(m1, 1, 0))
            d0 = _bc(n) + cum0 - 1
            d1 = _bc(n + cnt0) + cum1 - 1
            plsc.store_scatter(mylab, [d0], lab0, mask=m0)
            plsc.store_scatter(mypos, [d0], _bc(v2 * 32) + iota, mask=m0)
            plsc.store_scatter(mylab, [d1], lab1, mask=m1)
            plsc.store_scatter(mypos, [d1], _bc(v2 * 32 + 16) + iota, mask=m1)
            return n + cnt0 + cnt1

        return lax.cond(cnt0 + cnt1 > 0, do, lambda n: n, n)

    n = lax.fori_loop(0, B // 32, route_body, _i32(0))
    nv = (n + 15) // 16

    def flush16(bufb, col16, p16, fc):
        s = fc % 4

        def drain(_):
            pltpu.make_async_copy(rowstage.at[s], out.at[posstage.at[s]],
                                  ssem).wait()
            return 0

        lax.cond(fc >= 4, drain, lambda _: 0, 0)
        plsc.store_scatter(posstage, [_bc(s), iota], p16)
        for h in range(H):
            vals = plsc.load_gather(bufb, [_bc(_i32(h)), col16])
            plsc.store_scatter(rowstage, [_bc(s), iota, _bc(_i32(h))], vals)
        pltpu.async_copy(rowstage.at[s], out.at[posstage.at[s]], ssem)
        return fc + 1

    def extract_chunk(bufb, clo, state):
        def half(idxu, labg, lane_ok, m, cnt, st):
            wr, rd, fc = st

            def append(st):
                wr, rd, fc = st
                posv = plsc.load_gather(mypos, [idxu], mask=lane_ok)
                cum = plsc.cumsum(jnp.where(m, 1, 0))
                q = (_bc(wr) + cum - 1) % 32
                plsc.store_scatter(candcol, [q], labg - _bc(clo), mask=m)
                plsc.store_scatter(candpos, [q], posv, mask=m)
                wr = wr + cnt

                def do_flush(st):
                    wr, rd, fc = st
                    sel = (_bc(rd) + iota) % 32
                    col16 = plsc.load_gather(candcol, [sel])
                    p16 = plsc.load_gather(candpos, [sel])
                    fc = flush16(bufb, col16, p16, fc)
                    return (wr, rd + 16, fc)

                return lax.cond(wr - rd >= 16, do_flush, lambda st: st,
                                (wr, rd, fc))

            return lax.cond(cnt > 0, append, lambda st: st, (wr, rd, fc))

        def u_body(u, st):
            idx0 = _bc(u * 32) + iota
            idx1 = _bc(u * 32 + 16) + iota
            ok0 = idx0 < _bc(n)
            ok1 = idx1 < _bc(n)
            lab0 = plsc.load_gather(mylab, [idx0], mask=ok0)
            lab1 = plsc.load_gather(mylab, [idx1], mask=ok1)
            m0 = ok0 & (lab0 >= _bc(clo)) & (lab0 < _bc(clo + CHW))
            m1 = ok1 & (lab1 >= _bc(clo)) & (lab1 < _bc(clo + CHW))
            cnt0 = jnp.sum(jnp.where(m0, 1, 0))
            cnt1 = jnp.sum(jnp.where(m1, 1, 0))
            st = half(idx0, lab0, ok0, m0, cnt0, st)
            return half(idx1, lab1, ok1, m1, cnt1, st)

        state = lax.fori_loop(0, (n + 31) // 32, u_body, state)

        def tail(st):
            wr, rd, fc = st
            k = wr - rd
            sel = (_bc(rd) + iota) % 32
            col16 = plsc.load_gather(candcol, [sel])
            p16 = plsc.load_gather(candpos, [sel])
            c0 = plsc.load_gather(candcol, [_bc(rd % 32)])
            p0 = plsc.load_gather(candpos, [_bc(rd % 32)])
            col16 = jnp.where(iota < _bc(k), col16, c0)
            p16 = jnp.where(iota < _bc(k), p16, p0)
            fc = flush16(bufb, col16, p16, fc)
            return (wr, wr, fc)

        wr, rd, fc = state
        return lax.cond(wr - rd > 0, tail, lambda st: st, (wr, rd, fc))

    def chunk_step(g, bufb, semb, state):
        base = (sw + g) * CHW
        pltpu.make_async_copy(
            tT.at[:, pl.ds(pl.multiple_of(base, CHW), CHW)], bufb, semb).wait()

        state = extract_chunk(bufb, base, state)

        base2 = (sw + g + 2) * CHW

        @pl.when(g + 2 < nw)
        def _():
            pltpu.async_copy(
                tT.at[:, pl.ds(pl.multiple_of(base2, CHW), CHW)], bufb, semb)

        return state

    def go_body(go, state):
        g0 = go * 2
        state = lax.cond(g0 < nw,
                         lambda st: chunk_step(g0, buf0, sem0, st),
                         lambda st: st, state)
        return lax.cond(g0 + 1 < nw,
                        lambda st: chunk_step(g0 + 1, buf1, sem1, st),
                        lambda st: st, state)

    state = lax.fori_loop(0, GMAX // 2, go_body, (_i32(0), _i32(0), _i32(0)))

    def do_tail(state):
        pltpu.sync_copy(tail_in, buf0.at[:, pl.ds(0, 128)])
        return extract_chunk(buf0, _i32(TAILC), state)

    state = lax.cond(w == NW - 1, do_tail, lambda st: st, state)
    wr, rd, fc = state

    def fdrain(k, _):
        pltpu.make_async_copy(rowstage.at[k % 4],
                              out.at[posstage.at[k % 4]], ssem).wait()
        return 0

    lax.fori_loop(jnp.maximum(fc - 4, 0), fc, fdrain, 0)


def kernel(labels, embedding_table):
    tT = embedding_table.T
    lab = labels.astype(jnp.int32)
    tail = jnp.pad(embedding_table[TAILC:].T, ((0, 0), (0, 128 - (V - TAILC))))
    call = pl.kernel(
        _scan_body,
        out_type=jax.ShapeDtypeStruct((B, 128), jnp.float32),
        mesh=plsc.VectorSubcoreMesh(core_axis_name="c", subcore_axis_name="s"),
        scratch_types=[
            pltpu.VMEM((B,), jnp.int32),
            pltpu.VMEM((B,), jnp.int32),
            pltpu.VMEM((B,), jnp.int32),
            pltpu.VMEM((H, CHW), jnp.float32),
            pltpu.VMEM((H, CHW), jnp.float32),
            pltpu.VMEM((4, 16, 128), jnp.float32),
            pltpu.VMEM((4, 16), jnp.int32),
            pltpu.VMEM((32,), jnp.int32),
            pltpu.VMEM((32,), jnp.int32),
            pltpu.SemaphoreType.DMA,
            pltpu.SemaphoreType.DMA,
            pltpu.SemaphoreType.DMA,
        ],
        compiler_params=pltpu.CompilerParams(needs_layout_passes=False),
    )
    return call(tT, lab, tail)[:, :H]
